# baseline (speedup 1.0000x reference)
import functools

import jax
import jax.numpy as jnp
from jax import lax
from jax.experimental import pallas as pl
from jax.experimental.pallas import tpu as pltpu
from jax.experimental.pallas import tpu_sc as plsc

N_NODES = 50000
N_EDGES = 1600000

NC = 2
NS = 16
NW = NC * NS

CHUNK = 125
C_STAGE = 10
EPW_STEP = C_STAGE * CHUNK
N_OUT = 40
assert NW * N_OUT * EPW_STEP == N_EDGES
NB = N_EDGES // CHUNK
N_PAD = 50176
RPS = N_PAD // NS


def _make_sc_agg(F):
  mesh = plsc.VectorSubcoreMesh(core_axis_name="c", subcore_axis_name="s")

  @functools.partial(
      pl.kernel,
      out_type=jax.ShapeDtypeStruct((NC, N_PAD, F), jnp.float32),
      mesh=mesh,
      scratch_types=[
          pltpu.VMEM((C_STAGE, CHUNK), jnp.int32),
          pltpu.VMEM((C_STAGE, CHUNK), jnp.int32),
          pltpu.VMEM((C_STAGE, CHUNK, F), jnp.float32),
          pltpu.VMEM((C_STAGE, CHUNK), jnp.int32),
          pltpu.VMEM((C_STAGE, CHUNK), jnp.int32),
          pltpu.VMEM((C_STAGE, CHUNK, F), jnp.float32),
          pltpu.VMEM_SHARED((N_PAD, F), jnp.float32),
          pltpu.SemaphoreType.DMA,
          pltpu.SemaphoreType.DMA,
          pltpu.SemaphoreType.DMA,
          pltpu.SemaphoreType.DMA,
          pltpu.SemaphoreType.DMA,
          pltpu.SemaphoreType.DMA,
          pltpu.SemaphoreType.DMA,
          pltpu.SemaphoreType.DMA,
      ],
      compiler_params=pltpu.CompilerParams(use_tc_tiling_on_sc=False),
  )
  def agg(ins_hbm, ei_hbm, zeros_hbm, out_hbm,
          sidx0, didx0, rows0, sidx1, didx1, rows1, acc,
          gsem0, gsem1, ssem0, ssem1, isemS0, isemS1, isemD0, isemD1):
    c = lax.axis_index("c")
    s = lax.axis_index("s")
    wid = s * NC + c
    base = wid * N_OUT
    bufs = ((sidx0, didx0, rows0, gsem0, ssem0, isemS0, isemD0),
            (sidx1, didx1, rows1, gsem1, ssem1, isemS1, isemD1))

    pltpu.sync_copy(zeros_hbm, acc.at[pl.ds(s * RPS, RPS)])
    plsc.subcore_barrier()

    def prefetch_sidx(b, stage):
      sidx, _, _, _, _, isemS, _ = bufs[b]
      blk = (base + stage) * C_STAGE
      pltpu.async_copy(ei_hbm.at[0, pl.ds(blk, C_STAGE)], sidx, isemS)

    def fire(b, stage):
      sidx, didx, rows, gsem, _, isemS, isemD = bufs[b]
      blk = (base + stage) * C_STAGE
      pltpu.async_copy(ei_hbm.at[1, pl.ds(blk, C_STAGE)], didx, isemD)
      pltpu.make_async_copy(ei_hbm.at[0, pl.ds(blk, C_STAGE)], sidx,
                            isemS).wait()
      for j in range(C_STAGE):
        pltpu.async_copy(ins_hbm.at[sidx.at[j]], rows.at[j], gsem)

    def drain_gather(b):
      sidx, _, rows, gsem, _, _, _ = bufs[b]
      for j in range(C_STAGE):
        pltpu.make_async_copy(ins_hbm.at[sidx.at[j]], rows.at[j], gsem).wait()

    def scat(b, stage):
      _, didx, rows, _, ssem, _, isemD = bufs[b]
      blk = (base + stage) * C_STAGE
      pltpu.make_async_copy(ei_hbm.at[1, pl.ds(blk, C_STAGE)], didx,
                            isemD).wait()
      for j in range(C_STAGE):
        pltpu.async_copy(rows.at[j], acc.at[didx.at[j]], ssem, add=True)

    def drain_scatter(b):
      _, didx, rows, _, ssem, _, _ = bufs[b]
      for j in range(C_STAGE):
        pltpu.make_async_copy(rows.at[j], acc.at[didx.at[j]], ssem).wait()

    prefetch_sidx(0, 0)
    prefetch_sidx(1, 1)
    fire(0, 0)
    fire(1, 1)

    @pl.loop(0, N_OUT // 2)
    def _(i):
      more = i < N_OUT // 2 - 1
      drain_gather(0)

      @pl.when(more)
      def _():
        prefetch_sidx(0, 2 * i + 2)

      scat(0, 2 * i)
      drain_gather(1)

      @pl.when(more)
      def _():
        prefetch_sidx(1, 2 * i + 3)

      scat(1, 2 * i + 1)

      @pl.when(more)
      def _():
        drain_scatter(0)
        fire(0, 2 * i + 2)
        drain_scatter(1)
        fire(1, 2 * i + 3)

    drain_scatter(0)
    drain_scatter(1)

    plsc.subcore_barrier()
    pltpu.sync_copy(acc.at[pl.ds(s * RPS, RPS)],
                    out_hbm.at[c, pl.ds(s * RPS, RPS)])

  return agg


def _make_sc_degree():
  mesh = plsc.VectorSubcoreMesh(core_axis_name="c", subcore_axis_name="s")

  @functools.partial(
      pl.kernel,
      out_type=jax.ShapeDtypeStruct((NC, N_PAD, 16), jnp.float32),
      mesh=mesh,
      scratch_types=[
          pltpu.VMEM((C_STAGE, CHUNK), jnp.int32),
          pltpu.VMEM((C_STAGE, CHUNK), jnp.int32),
          pltpu.VMEM((CHUNK, 16), jnp.float32),
          pltpu.VMEM_SHARED((N_PAD, 16), jnp.float32),
          pltpu.SemaphoreType.DMA,
          pltpu.SemaphoreType.DMA,
      ],
      compiler_params=pltpu.CompilerParams(use_tc_tiling_on_sc=False),
  )
  def deg(ei_hbm, ones_hbm, zeros_hbm, out_hbm,
          didx0, didx1, ones_v, acc, ssem0, ssem1):
    c = lax.axis_index("c")
    s = lax.axis_index("s")
    wid = s * NC + c
    base = wid * N_OUT
    bufs = ((didx0, ssem0), (didx1, ssem1))

    pltpu.sync_copy(ones_hbm, ones_v)
    pltpu.sync_copy(zeros_hbm, acc.at[pl.ds(s * RPS, RPS)])
    plsc.subcore_barrier()

    def fire(b, stage):
      didx, ssem = bufs[b]
      blk = (base + stage) * C_STAGE
      pltpu.sync_copy(ei_hbm.at[1, pl.ds(blk, C_STAGE)], didx)
      for j in range(C_STAGE):
        pltpu.async_copy(ones_v, acc.at[didx.at[j]], ssem, add=True)

    def drain(b):
      didx, ssem = bufs[b]
      for j in range(C_STAGE):
        pltpu.make_async_copy(ones_v, acc.at[didx.at[j]], ssem).wait()

    fire(0, 0)
    fire(1, 1)

    @pl.loop(1, N_OUT // 2)
    def _(i):
      drain(0)
      fire(0, 2 * i)
      drain(1)
      fire(1, 2 * i + 1)

    drain(0)
    drain(1)

    plsc.subcore_barrier()
    pltpu.sync_copy(acc.at[pl.ds(s * RPS, RPS)],
                    out_hbm.at[c, pl.ds(s * RPS, RPS)])

  return deg


NP8 = N_PAD // 8
_ROWS = 784
_GRID = NP8 // _ROWS


def _rowspec(f, r=_ROWS):
  return pl.BlockSpec((r, f), lambda i: (i, 0))


def _aggspec(f, r=_ROWS):
  return pl.BlockSpec((NC, r, f), lambda i: (0, i, 0))


def _fullspec(shape):
  return pl.BlockSpec(shape, lambda i: (0,) * len(shape))


import numpy as np

_SEL_A = np.zeros((256, 128), np.float32)
_SEL_B = np.zeros((256, 128), np.float32)
for _k in range(8):
  for _j in range(16):
    _SEL_A[32 * _k + _j, 16 * _k + _j] = 1.0
    _SEL_B[32 * _k + 16 + _j, 16 * _k + _j] = 1.0
_SWAP01 = np.eye(128, dtype=np.float32)
for _k in range(8):
  _SWAP01[16 * _k, 16 * _k] = 0.0
  _SWAP01[16 * _k + 1, 16 * _k + 1] = 0.0
  _SWAP01[16 * _k, 16 * _k + 1] = 1.0
  _SWAP01[16 * _k + 1, 16 * _k] = 1.0


def _kron8(w):
  ki, ko = w.shape
  eye = jnp.eye(8, dtype=jnp.float32)
  return (eye[:, None, :, None] * w[None, :, None, :]).reshape(
      8 * ki, 8 * ko)


def _tc_prep(cnt_ref, x_ref, s0_ref, dis_ref):
  dis = lax.rsqrt(cnt_ref[0] + cnt_ref[1] + 1.0)
  dis_ref[...] = dis
  s0_ref[...] = x_ref[...] * dis


def _tc_layer1(a_ref, s0_ref, dis_ref, w18_ref, b18_ref, sa_ref, sb_ref,
               s1a_ref, s1b_ref):
  dis = dis_ref[...]
  pre = dis * (a_ref[0] + a_ref[1] + s0_ref[...])
  h = jnp.dot(pre, w18_ref[...], preferred_element_type=jnp.float32)
  h = jnp.maximum(h + b18_ref[...], 0.0)
  s1a_ref[...] = jnp.dot(h, sa_ref[...],
                         preferred_element_type=jnp.float32) * dis
  s1b_ref[...] = jnp.dot(h, sb_ref[...],
                         preferred_element_type=jnp.float32) * dis


def _tc_layer2(g0_ref, g1_ref, s1a_ref, s1b_ref, dis_ref, w2a8_ref, w2b8_ref,
               b28_ref, w38_ref, ts_ref):
  dis = dis_ref[...]
  pre_a = dis * (g0_ref[0] + g0_ref[1] + s1a_ref[...])
  pre_b = dis * (g1_ref[0] + g1_ref[1] + s1b_ref[...])
  h = (jnp.dot(pre_a, w2a8_ref[...], preferred_element_type=jnp.float32)
       + jnp.dot(pre_b, w2b8_ref[...], preferred_element_type=jnp.float32))
  h = jnp.maximum(h + b28_ref[...], 0.0)
  t = jnp.dot(h, w38_ref[...], preferred_element_type=jnp.float32)
  ts_ref[...] = t * dis


def _tc_final(a_ref, ts_ref, dis_ref, b3p_ref, swap_ref, out_ref):
  o = dis_ref[...] * (a_ref[0] + a_ref[1] + ts_ref[...]) + b3p_ref[...]
  ow = jnp.dot(o, swap_ref[...], preferred_element_type=jnp.float32)
  m = jnp.maximum(o, ow)
  lse = m + jnp.log(jnp.exp(o - m) + jnp.exp(ow - m))
  out_ref[...] = o - lse


def kernel(x, edge_index, W1, b1, W2, b2, W3, b3):
  ei3 = edge_index.astype(jnp.int32).reshape(2, NB, CHUNK)

  x_p = jnp.concatenate(
      [x.reshape(N_NODES // 8, 128),
       jnp.zeros((NP8 - N_NODES // 8, 128), jnp.float32)])

  zeros16 = jnp.zeros((RPS, 16), jnp.float32)
  ones16 = jnp.ones((CHUNK, 16), jnp.float32)

  def packed(a):
    return a.reshape(NC, NP8, 128)

  def unpacked(a):
    return a.reshape(N_PAD, 16)

  counts = packed(_make_sc_degree()(ei3, ones16, zeros16))

  s0_p, dis_p = pl.pallas_call(
      _tc_prep,
      grid=(_GRID,),
      in_specs=[_aggspec(128), _rowspec(128)],
      out_specs=[_rowspec(128), _rowspec(128)],
      out_shape=[
          jax.ShapeDtypeStruct((NP8, 128), jnp.float32),
          jax.ShapeDtypeStruct((NP8, 128), jnp.float32),
      ],
  )(counts, x_p)

  agg1 = packed(_make_sc_agg(16)(unpacked(s0_p), ei3, zeros16))

  w18 = _kron8(W1)
  b18 = jnp.tile(b1, 8).reshape(1, 256)
  sel_a = jnp.asarray(_SEL_A)
  sel_b = jnp.asarray(_SEL_B)

  s1a_p, s1b_p = pl.pallas_call(
      _tc_layer1,
      grid=(_GRID,),
      in_specs=[_aggspec(128), _rowspec(128), _rowspec(128),
                _fullspec((128, 256)), _fullspec((1, 256)),
                _fullspec((256, 128)), _fullspec((256, 128))],
      out_specs=[_rowspec(128), _rowspec(128)],
      out_shape=[
          jax.ShapeDtypeStruct((NP8, 128), jnp.float32),
          jax.ShapeDtypeStruct((NP8, 128), jnp.float32),
      ],
  )(agg1, s0_p, dis_p, w18, b18, sel_a, sel_b)

  agg2h0 = packed(_make_sc_agg(16)(unpacked(s1a_p), ei3, zeros16))
  agg2h1 = packed(_make_sc_agg(16)(unpacked(s1b_p), ei3, zeros16))

  w2a8 = _kron8(W2[:16])
  w2b8 = _kron8(W2[16:])
  b28 = jnp.tile(b2, 8).reshape(1, 512)
  w38 = _kron8(jnp.pad(W3, ((0, 0), (0, 14))))

  ts_p = pl.pallas_call(
      _tc_layer2,
      grid=(_GRID,),
      in_specs=[_aggspec(128), _aggspec(128), _rowspec(128), _rowspec(128),
                _rowspec(128), _fullspec((128, 512)), _fullspec((128, 512)),
                _fullspec((1, 512)), _fullspec((512, 128))],
      out_specs=_rowspec(128),
      out_shape=jax.ShapeDtypeStruct((NP8, 128), jnp.float32),
  )(agg2h0, agg2h1, s1a_p, s1b_p, dis_p, w2a8, w2b8, b28, w38)

  agg3 = packed(_make_sc_agg(16)(unpacked(ts_p), ei3, zeros16))

  b3p = jnp.tile(jnp.concatenate([b3, jnp.zeros((14,), jnp.float32)]),
                 8).reshape(1, 128)
  out_p = pl.pallas_call(
      _tc_final,
      grid=(_GRID,),
      in_specs=[_aggspec(128), _rowspec(128), _rowspec(128),
                _fullspec((1, 128)), _fullspec((128, 128))],
      out_specs=_rowspec(128),
      out_shape=jax.ShapeDtypeStruct((NP8, 128), jnp.float32),
  )(agg3, ts_p, dis_p, b3p, jnp.asarray(_SWAP01))

  return out_p.reshape(N_PAD, 16)[:N_NODES, :2]


# speedup vs baseline: 1.0176x; 1.0176x over previous
import functools

import jax
import jax.numpy as jnp
from jax import lax
from jax.experimental import pallas as pl
from jax.experimental.pallas import tpu as pltpu
from jax.experimental.pallas import tpu_sc as plsc

N_NODES = 50000
N_EDGES = 1600000

NC = 2
NS = 16
NW = NC * NS

CHUNK = 125
C_STAGE = 10
EPW_STEP = C_STAGE * CHUNK
N_OUT = 40
assert NW * N_OUT * EPW_STEP == N_EDGES
NB = N_EDGES // CHUNK
N_PAD = 50176
RPS = N_PAD // NS


def _make_sc_agg(F):
  mesh = plsc.VectorSubcoreMesh(core_axis_name="c", subcore_axis_name="s")

  @functools.partial(
      pl.kernel,
      out_type=jax.ShapeDtypeStruct((NC, N_PAD, F), jnp.float32),
      mesh=mesh,
      scratch_types=[
          pltpu.VMEM((C_STAGE, CHUNK), jnp.int32),
          pltpu.VMEM((C_STAGE, CHUNK), jnp.int32),
          pltpu.VMEM((C_STAGE, CHUNK, F), jnp.float32),
          pltpu.VMEM((C_STAGE, CHUNK), jnp.int32),
          pltpu.VMEM((C_STAGE, CHUNK), jnp.int32),
          pltpu.VMEM((C_STAGE, CHUNK, F), jnp.float32),
          pltpu.VMEM_SHARED((N_PAD, F), jnp.float32),
          pltpu.SemaphoreType.DMA,
          pltpu.SemaphoreType.DMA,
          pltpu.SemaphoreType.DMA,
          pltpu.SemaphoreType.DMA,
          pltpu.SemaphoreType.DMA,
          pltpu.SemaphoreType.DMA,
          pltpu.SemaphoreType.DMA,
          pltpu.SemaphoreType.DMA,
      ],
      compiler_params=pltpu.CompilerParams(use_tc_tiling_on_sc=False),
  )
  def agg(ins_hbm, ei_hbm, zeros_hbm, out_hbm,
          sidx0, didx0, rows0, sidx1, didx1, rows1, acc,
          gsem0, gsem1, ssem0, ssem1, isemS0, isemS1, isemD0, isemD1):
    c = lax.axis_index("c")
    s = lax.axis_index("s")
    wid = s * NC + c
    base = wid * N_OUT
    bufs = ((sidx0, didx0, rows0, gsem0, ssem0, isemS0, isemD0),
            (sidx1, didx1, rows1, gsem1, ssem1, isemS1, isemD1))

    pltpu.sync_copy(zeros_hbm, acc.at[pl.ds(s * RPS, RPS)])
    plsc.subcore_barrier()

    def fire(b, stage):
      sidx, didx, rows, gsem, _, isemS, isemD = bufs[b]
      blk = (base + stage) * C_STAGE
      pltpu.async_copy(ei_hbm.at[0, pl.ds(blk, C_STAGE)], sidx, isemS)
      pltpu.async_copy(ei_hbm.at[1, pl.ds(blk, C_STAGE)], didx, isemD)
      pltpu.make_async_copy(ei_hbm.at[0, pl.ds(blk, C_STAGE)], sidx,
                            isemS).wait()
      for j in range(C_STAGE):
        pltpu.async_copy(ins_hbm.at[sidx.at[j]], rows.at[j], gsem)

    def drain_gather(b):
      sidx, _, rows, gsem, _, _, _ = bufs[b]
      for j in range(C_STAGE):
        pltpu.make_async_copy(ins_hbm.at[sidx.at[j]], rows.at[j], gsem).wait()

    def scat(b, stage):
      _, didx, rows, _, ssem, _, isemD = bufs[b]
      blk = (base + stage) * C_STAGE
      pltpu.make_async_copy(ei_hbm.at[1, pl.ds(blk, C_STAGE)], didx,
                            isemD).wait()
      for j in range(C_STAGE):
        pltpu.async_copy(rows.at[j], acc.at[didx.at[j]], ssem, add=True)

    def drain_scatter(b):
      _, didx, rows, _, ssem, _, _ = bufs[b]
      for j in range(C_STAGE):
        pltpu.make_async_copy(rows.at[j], acc.at[didx.at[j]], ssem).wait()

    fire(0, 0)
    fire(1, 1)

    @pl.loop(0, N_OUT // 2)
    def _(i):
      drain_gather(0)
      scat(0, 2 * i)
      drain_gather(1)
      scat(1, 2 * i + 1)

      @pl.when(i < N_OUT // 2 - 1)
      def _():
        drain_scatter(0)
        fire(0, 2 * i + 2)
        drain_scatter(1)
        fire(1, 2 * i + 3)

    drain_scatter(0)
    drain_scatter(1)

    plsc.subcore_barrier()
    pltpu.sync_copy(acc.at[pl.ds(s * RPS, RPS)],
                    out_hbm.at[c, pl.ds(s * RPS, RPS)])

  return agg


def _make_sc_degree():
  mesh = plsc.VectorSubcoreMesh(core_axis_name="c", subcore_axis_name="s")

  @functools.partial(
      pl.kernel,
      out_type=jax.ShapeDtypeStruct((NC, N_PAD, 16), jnp.float32),
      mesh=mesh,
      scratch_types=[
          pltpu.VMEM((C_STAGE, CHUNK), jnp.int32),
          pltpu.VMEM((C_STAGE, CHUNK), jnp.int32),
          pltpu.VMEM((CHUNK, 16), jnp.float32),
          pltpu.VMEM_SHARED((N_PAD, 16), jnp.float32),
          pltpu.SemaphoreType.DMA,
          pltpu.SemaphoreType.DMA,
      ],
      compiler_params=pltpu.CompilerParams(use_tc_tiling_on_sc=False),
  )
  def deg(ei_hbm, ones_hbm, zeros_hbm, out_hbm,
          didx0, didx1, ones_v, acc, ssem0, ssem1):
    c = lax.axis_index("c")
    s = lax.axis_index("s")
    wid = s * NC + c
    base = wid * N_OUT
    bufs = ((didx0, ssem0), (didx1, ssem1))

    pltpu.sync_copy(ones_hbm, ones_v)
    pltpu.sync_copy(zeros_hbm, acc.at[pl.ds(s * RPS, RPS)])
    plsc.subcore_barrier()

    def fire(b, stage):
      didx, ssem = bufs[b]
      blk = (base + stage) * C_STAGE
      pltpu.sync_copy(ei_hbm.at[1, pl.ds(blk, C_STAGE)], didx)
      for j in range(C_STAGE):
        pltpu.async_copy(ones_v, acc.at[didx.at[j]], ssem, add=True)

    def drain(b):
      didx, ssem = bufs[b]
      for j in range(C_STAGE):
        pltpu.make_async_copy(ones_v, acc.at[didx.at[j]], ssem).wait()

    fire(0, 0)
    fire(1, 1)

    @pl.loop(1, N_OUT // 2)
    def _(i):
      drain(0)
      fire(0, 2 * i)
      drain(1)
      fire(1, 2 * i + 1)

    drain(0)
    drain(1)

    plsc.subcore_barrier()
    pltpu.sync_copy(acc.at[pl.ds(s * RPS, RPS)],
                    out_hbm.at[c, pl.ds(s * RPS, RPS)])

  return deg


NP8 = N_PAD // 8
_ROWS = 784
_GRID = NP8 // _ROWS


def _rowspec(f, r=_ROWS):
  return pl.BlockSpec((r, f), lambda i: (i, 0))


def _aggspec(f, r=_ROWS):
  return pl.BlockSpec((NC, r, f), lambda i: (0, i, 0))


def _fullspec(shape):
  return pl.BlockSpec(shape, lambda i: (0,) * len(shape))


import numpy as np

_SEL_A = np.zeros((256, 128), np.float32)
_SEL_B = np.zeros((256, 128), np.float32)
for _k in range(8):
  for _j in range(16):
    _SEL_A[32 * _k + _j, 16 * _k + _j] = 1.0
    _SEL_B[32 * _k + 16 + _j, 16 * _k + _j] = 1.0
_SWAP01 = np.eye(128, dtype=np.float32)
for _k in range(8):
  _SWAP01[16 * _k, 16 * _k] = 0.0
  _SWAP01[16 * _k + 1, 16 * _k + 1] = 0.0
  _SWAP01[16 * _k, 16 * _k + 1] = 1.0
  _SWAP01[16 * _k + 1, 16 * _k] = 1.0


def _kron8(w):
  ki, ko = w.shape
  eye = jnp.eye(8, dtype=jnp.float32)
  return (eye[:, None, :, None] * w[None, :, None, :]).reshape(
      8 * ki, 8 * ko)


def _tc_prep(cnt_ref, x_ref, s0_ref, dis_ref):
  dis = lax.rsqrt(cnt_ref[0] + cnt_ref[1] + 1.0)
  dis_ref[...] = dis
  s0_ref[...] = x_ref[...] * dis


def _tc_layer1(a_ref, s0_ref, dis_ref, w18_ref, b18_ref, sa_ref, sb_ref,
               s1a_ref, s1b_ref):
  dis = dis_ref[...]
  pre = dis * (a_ref[0] + a_ref[1] + s0_ref[...])
  h = jnp.dot(pre, w18_ref[...], preferred_element_type=jnp.float32)
  h = jnp.maximum(h + b18_ref[...], 0.0)
  s1a_ref[...] = jnp.dot(h, sa_ref[...],
                         preferred_element_type=jnp.float32) * dis
  s1b_ref[...] = jnp.dot(h, sb_ref[...],
                         preferred_element_type=jnp.float32) * dis


def _tc_layer2(g0_ref, g1_ref, s1a_ref, s1b_ref, dis_ref, w2a8_ref, w2b8_ref,
               b28_ref, w38_ref, ts_ref):
  dis = dis_ref[...]
  pre_a = dis * (g0_ref[0] + g0_ref[1] + s1a_ref[...])
  pre_b = dis * (g1_ref[0] + g1_ref[1] + s1b_ref[...])
  h = (jnp.dot(pre_a, w2a8_ref[...], preferred_element_type=jnp.float32)
       + jnp.dot(pre_b, w2b8_ref[...], preferred_element_type=jnp.float32))
  h = jnp.maximum(h + b28_ref[...], 0.0)
  t = jnp.dot(h, w38_ref[...], preferred_element_type=jnp.float32)
  ts_ref[...] = t * dis


def _tc_final(a_ref, ts_ref, dis_ref, b3p_ref, swap_ref, out_ref):
  o = dis_ref[...] * (a_ref[0] + a_ref[1] + ts_ref[...]) + b3p_ref[...]
  ow = jnp.dot(o, swap_ref[...], preferred_element_type=jnp.float32)
  m = jnp.maximum(o, ow)
  lse = m + jnp.log(jnp.exp(o - m) + jnp.exp(ow - m))
  out_ref[...] = o - lse


def kernel(x, edge_index, W1, b1, W2, b2, W3, b3):
  ei3 = edge_index.astype(jnp.int32).reshape(2, NB, CHUNK)

  x_p = jnp.concatenate(
      [x.reshape(N_NODES // 8, 128),
       jnp.zeros((NP8 - N_NODES // 8, 128), jnp.float32)])

  zeros16 = jnp.zeros((RPS, 16), jnp.float32)
  ones16 = jnp.ones((CHUNK, 16), jnp.float32)

  def packed(a):
    return a.reshape(NC, NP8, 128)

  def unpacked(a):
    return a.reshape(N_PAD, 16)

  counts = packed(_make_sc_degree()(ei3, ones16, zeros16))

  s0_p, dis_p = pl.pallas_call(
      _tc_prep,
      grid=(_GRID,),
      in_specs=[_aggspec(128), _rowspec(128)],
      out_specs=[_rowspec(128), _rowspec(128)],
      out_shape=[
          jax.ShapeDtypeStruct((NP8, 128), jnp.float32),
          jax.ShapeDtypeStruct((NP8, 128), jnp.float32),
      ],
  )(counts, x_p)

  agg1 = packed(_make_sc_agg(16)(unpacked(s0_p), ei3, zeros16))

  w18 = _kron8(W1)
  b18 = jnp.tile(b1, 8).reshape(1, 256)
  sel_a = jnp.asarray(_SEL_A)
  sel_b = jnp.asarray(_SEL_B)

  s1a_p, s1b_p = pl.pallas_call(
      _tc_layer1,
      grid=(_GRID,),
      in_specs=[_aggspec(128), _rowspec(128), _rowspec(128),
                _fullspec((128, 256)), _fullspec((1, 256)),
                _fullspec((256, 128)), _fullspec((256, 128))],
      out_specs=[_rowspec(128), _rowspec(128)],
      out_shape=[
          jax.ShapeDtypeStruct((NP8, 128), jnp.float32),
          jax.ShapeDtypeStruct((NP8, 128), jnp.float32),
      ],
  )(agg1, s0_p, dis_p, w18, b18, sel_a, sel_b)

  agg2h0 = packed(_make_sc_agg(16)(unpacked(s1a_p), ei3, zeros16))
  agg2h1 = packed(_make_sc_agg(16)(unpacked(s1b_p), ei3, zeros16))

  w2a8 = _kron8(W2[:16])
  w2b8 = _kron8(W2[16:])
  b28 = jnp.tile(b2, 8).reshape(1, 512)
  w38 = _kron8(jnp.pad(W3, ((0, 0), (0, 14))))

  ts_p = pl.pallas_call(
      _tc_layer2,
      grid=(_GRID,),
      in_specs=[_aggspec(128), _aggspec(128), _rowspec(128), _rowspec(128),
                _rowspec(128), _fullspec((128, 512)), _fullspec((128, 512)),
                _fullspec((1, 512)), _fullspec((512, 128))],
      out_specs=_rowspec(128),
      out_shape=jax.ShapeDtypeStruct((NP8, 128), jnp.float32),
  )(agg2h0, agg2h1, s1a_p, s1b_p, dis_p, w2a8, w2b8, b28, w38)

  agg3 = packed(_make_sc_agg(16)(unpacked(ts_p), ei3, zeros16))

  b3p = jnp.tile(jnp.concatenate([b3, jnp.zeros((14,), jnp.float32)]),
                 8).reshape(1, 128)
  out_p = pl.pallas_call(
      _tc_final,
      grid=(_GRID,),
      in_specs=[_aggspec(128), _rowspec(128), _rowspec(128),
                _fullspec((1, 128)), _fullspec((128, 128))],
      out_specs=_rowspec(128),
      out_shape=jax.ShapeDtypeStruct((NP8, 128), jnp.float32),
  )(agg3, ts_p, dis_p, b3p, jnp.asarray(_SWAP01))

  return out_p.reshape(N_PAD, 16)[:N_NODES, :2]
